# Initial kernel scaffold; baseline (speedup 1.0000x reference)
import jax
import jax.numpy as jnp
from jax.experimental import pallas as pl


def kernel(feats1, feats2, positive_pairs):
    raise NotImplementedError("write your pallas kernel here")


# speedup vs baseline: 120.0881x; 120.0881x over previous
import functools

import jax
import jax.numpy as jnp
from jax import lax
from jax.experimental import pallas as pl
from jax.experimental.pallas import tpu as pltpu
from jax.experimental.pallas import tpu_sc as plsc

_C = 96
_H = 384
_W = 384
_NPIX = _H * _W
_NQ = 64
_K = 5
_BLK = 4096
_NBLK = _NPIX // _BLK
_NC = 2
_NS = 16
_QPW = _NQ // (_NC * _NS)


def _sc_gather_body(f1v_hbm, rowidx_hbm, out_hbm, idx_v, rows_v, sem):
    wid = lax.axis_index("s") * _NC + lax.axis_index("c")
    for jq in range(_QPW):
        q = wid * _QPW + jq
        pltpu.sync_copy(rowidx_hbm.at[q], idx_v)
        pltpu.async_copy(f1v_hbm.at[idx_v], rows_v, sem).wait()
        pltpu.sync_copy(rows_v, out_hbm.at[q])


def _gather_queries(f1v, rowidx):
    mesh = plsc.VectorSubcoreMesh(core_axis_name="c", subcore_axis_name="s")
    fn = functools.partial(
        pl.kernel,
        mesh=mesh,
        out_type=jax.ShapeDtypeStruct((_NQ, _C, 128), jnp.float32),
        scratch_types=[
            pltpu.VMEM((_C,), jnp.int32),
            pltpu.VMEM((_C, 128), jnp.float32),
            pltpu.SemaphoreType.DMA,
        ],
    )(_sc_gather_body)
    return fn(f1v, rowidx)


def _tc_body(q_ref, f2_ref, out_ref, top_ref):
    j = pl.program_id(0)

    @pl.when(j == 0)
    def _init():
        top_ref[...] = jnp.full((_NQ, _K), jnp.inf, dtype=jnp.float32)

    q = q_ref[...]
    f2 = f2_ref[...]
    dot = lax.dot_general(q, f2, (((1,), (0,)), ((), ())),
                          precision=lax.Precision.HIGHEST)
    qn = jnp.sum(q * q, axis=1, keepdims=True)
    fn = jnp.sum(f2 * f2, axis=0, keepdims=True)
    dist = jnp.maximum(qn - 2.0 * dot + fn, 0.0)

    mins = []
    for k in range(_K):
        m = jnp.min(dist, axis=1, keepdims=True)
        mins.append(m)
        if k < _K - 1:
            dist = jnp.where(dist == m, jnp.inf, dist)

    comb = jnp.concatenate([top_ref[...]] + mins, axis=1)
    news = []
    for k in range(_K):
        m = jnp.min(comb, axis=1, keepdims=True)
        news.append(m)
        if k < _K - 1:
            comb = jnp.where(comb == m, jnp.inf, comb)
    top_ref[...] = jnp.concatenate(news, axis=1)

    @pl.when(j == _NBLK - 1)
    def _fin():
        s = jnp.sum(top_ref[...], axis=1, keepdims=True) / float(_NQ * _K)
        out_ref[...] = jnp.sum(s, axis=0, keepdims=True)


def _topk_mean(q_all, f2_flat):
    return pl.pallas_call(
        _tc_body,
        grid=(_NBLK,),
        in_specs=[
            pl.BlockSpec((_NQ, _C), lambda j: (0, 0)),
            pl.BlockSpec((_C, _BLK), lambda j: (0, j)),
        ],
        out_specs=pl.BlockSpec((1, 1), lambda j: (0, 0)),
        out_shape=jax.ShapeDtypeStruct((1, 1), jnp.float32),
        scratch_shapes=[pltpu.VMEM((_NQ, _K), jnp.float32)],
    )(q_all, f2_flat)


def kernel(feats1, feats2, positive_pairs):
    p = positive_pairs[0, :, 0].astype(jnp.int32)
    ch = jnp.arange(_C, dtype=jnp.int32)
    rowidx = ch[None, :] * (_NPIX // 128) + (p // 128)[:, None]

    f1v = feats1.reshape(_C * _NPIX // 128, 128)
    rows = _gather_queries(f1v, rowidx)
    lane = (p % 128).astype(jnp.int32)[:, None, None]
    q_all = jnp.take_along_axis(rows, lane, axis=2)[..., 0]

    f2_flat = feats2.reshape(_C, _NPIX)
    out = _topk_mean(q_all, f2_flat)
    return out[0, 0]


# speedup vs baseline: 171.7056x; 1.4298x over previous
import functools

import jax
import jax.numpy as jnp
from jax import lax
from jax.experimental import pallas as pl
from jax.experimental.pallas import tpu as pltpu
from jax.experimental.pallas import tpu_sc as plsc

_C = 96
_H = 384
_W = 384
_NPIX = _H * _W
_NQ = 64
_K = 5
_BH = 16
_NBLK = _H // _BH
_NC = 2
_NS = 16
_QPW = _NQ // (_NC * _NS)


def _sc_gather_body(f1rows_hbm, rowidx_hbm, out_hbm, idx_v, rows_v, sem):
    wid = lax.axis_index("s") * _NC + lax.axis_index("c")
    for jq in range(_QPW):
        q = wid * _QPW + jq
        pltpu.sync_copy(rowidx_hbm.at[q], idx_v)
        pltpu.async_copy(f1rows_hbm.at[idx_v], rows_v, sem).wait()
        pltpu.sync_copy(rows_v, out_hbm.at[q])


def _gather_queries(f1rows, rowidx):
    mesh = plsc.VectorSubcoreMesh(core_axis_name="c", subcore_axis_name="s")
    fn = functools.partial(
        pl.kernel,
        mesh=mesh,
        out_type=jax.ShapeDtypeStruct((_NQ, _C, _W), jnp.float32),
        scratch_types=[
            pltpu.VMEM((_C,), jnp.int32),
            pltpu.VMEM((_C, _W), jnp.float32),
            pltpu.SemaphoreType.DMA,
        ],
    )(_sc_gather_body)
    return fn(f1rows, rowidx)


def _tc_body(q_ref, f2_ref, out_ref, top_ref):
    j = pl.program_id(0)

    @pl.when(j == 0)
    def _init():
        top_ref[...] = jnp.full((_NQ, _K), jnp.inf, dtype=jnp.float32)

    q = q_ref[...]
    qn = jnp.sum(q * q, axis=1, keepdims=True)

    strips = []
    for b in range(_BH):
        s2 = f2_ref[:, b, :]
        dot = lax.dot_general(q, s2, (((1,), (0,)), ((), ())),
                              precision=lax.Precision.HIGHEST)
        fn = jnp.sum(s2 * s2, axis=0, keepdims=True)
        strips.append(jnp.maximum(qn - 2.0 * dot + fn, 0.0))

    mins = []
    for k in range(_K):
        sm = [jnp.min(s, axis=1, keepdims=True) for s in strips]
        m = functools.reduce(jnp.minimum, sm)
        mins.append(m)
        if k < _K - 1:
            strips = [jnp.where(s == m, jnp.inf, s) for s in strips]

    comb = jnp.concatenate([top_ref[...]] + mins, axis=1)
    news = []
    for k in range(_K):
        m = jnp.min(comb, axis=1, keepdims=True)
        news.append(m)
        if k < _K - 1:
            comb = jnp.where(comb == m, jnp.inf, comb)
    top_ref[...] = jnp.concatenate(news, axis=1)

    @pl.when(j == _NBLK - 1)
    def _fin():
        s = jnp.sum(top_ref[...], axis=1, keepdims=True) / float(_NQ * _K)
        out_ref[...] = jnp.sum(s, axis=0, keepdims=True)


def _topk_mean(q_all, f2_3d):
    return pl.pallas_call(
        _tc_body,
        grid=(_NBLK,),
        in_specs=[
            pl.BlockSpec((_NQ, _C), lambda j: (0, 0)),
            pl.BlockSpec((_C, _BH, _W), lambda j: (0, j, 0)),
        ],
        out_specs=pl.BlockSpec((1, 1), lambda j: (0, 0)),
        out_shape=jax.ShapeDtypeStruct((1, 1), jnp.float32),
        scratch_shapes=[pltpu.VMEM((_NQ, _K), jnp.float32)],
    )(q_all, f2_3d)


def kernel(feats1, feats2, positive_pairs):
    p = positive_pairs[0, :, 0].astype(jnp.int32)
    ch = jnp.arange(_C, dtype=jnp.int32)
    rowidx = ch[None, :] * _H + (p // _W)[:, None]

    f1rows = feats1.reshape(_C * _H, _W)
    rows = _gather_queries(f1rows, rowidx)
    lane = (p % _W).astype(jnp.int32)[:, None, None]
    q_all = jnp.take_along_axis(rows, lane, axis=2)[..., 0]

    f2_3d = feats2.reshape(_C, _H, _W)
    out = _topk_mean(q_all, f2_3d)
    return out[0, 0]
